# baseline (speedup 1.0000x reference)
import functools

import jax
import jax.numpy as jnp
from jax import lax
from jax.experimental import pallas as pl
from jax.experimental.pallas import tpu as pltpu
from jax.experimental.pallas import tpu_sc as plsc

N_FEAT = 128
N_GATES = 64
MAX_CONN = N_FEAT + 2 + N_GATES
CHUNKS = 13
SUB = 8
LANES = 256
BLK = SUB * LANES


def _topk_sc_body(nc, gpw, gw_hbm, idx_hbm, gw_v, out_v):
    wid = lax.axis_index("s") * nc + lax.axis_index("c")
    big = jnp.int32(1 << 30)
    lane = lax.iota(jnp.int32, 16)

    nw = N_GATES // gpw

    @pl.when(wid < nw)
    def _body():
        _topk_sc_run(gpw, wid, big, lane, gw_hbm, idx_hbm, gw_v, out_v)


def _topk_sc_run(gpw, wid, big, lane, gw_hbm, idx_hbm, gw_v, out_v):
    pltpu.sync_copy(gw_hbm.at[pl.ds(wid * gpw, gpw)], gw_v)

    for j in range(gpw):
        g = wid * gpw + j
        n_valid = N_FEAT + 2 + g

        xs, cols, oks = [], [], []
        for c in range(CHUNKS):
            off = c * 16 if c < 12 else MAX_CONN - 16
            col = lane + off
            ok = (col < n_valid) if c < 12 else ((col >= 192) & (col < n_valid))
            xs.append(gw_v[j, pl.ds(off, 16)])
            cols.append(col)
            oks.append(ok)

        mv = jnp.where(oks[0], xs[0], -1e30)
        for c in range(1, CHUNKS):
            mv = jnp.maximum(mv, jnp.where(oks[c], xs[c], -1e30))
        m = jnp.max(mv)

        es = [jnp.where(oks[c], jnp.exp(xs[c] - m), 0.0) for c in range(CHUNKS)]
        sv = es[0]
        for c in range(1, CHUNKS):
            sv = sv + es[c]
        s = jnp.sum(sv)

        ps = [es[c] / s for c in range(CHUNKS)]

        m1v = jnp.where(oks[0], ps[0], -1.0)
        for c in range(1, CHUNKS):
            m1v = jnp.maximum(m1v, jnp.where(oks[c], ps[c], -1.0))
        m1 = jnp.max(m1v)

        i1v = jnp.where(oks[0] & (ps[0] == m1), cols[0], big)
        for c in range(1, CHUNKS):
            i1v = jnp.minimum(i1v, jnp.where(oks[c] & (ps[c] == m1), cols[c], big))
        i1 = jnp.min(i1v)

        m2v = jnp.where(oks[0] & (cols[0] != i1), ps[0], -1.0)
        for c in range(1, CHUNKS):
            m2v = jnp.maximum(m2v, jnp.where(oks[c] & (cols[c] != i1), ps[c], -1.0))
        m2 = jnp.max(m2v)

        i2v = jnp.where(oks[0] & (cols[0] != i1) & (ps[0] == m2), cols[0], big)
        for c in range(1, CHUNKS):
            i2v = jnp.minimum(
                i2v, jnp.where(oks[c] & (cols[c] != i1) & (ps[c] == m2), cols[c], big))
        i2 = jnp.min(i2v)

        out_v[j, :] = jnp.where(lane == 0, i1, jnp.where(lane == 1, i2, 0))

    pltpu.sync_copy(out_v, idx_hbm.at[pl.ds(wid * gpw, gpw)])


NSPLIT = 2


def _fused_chain_kernel(n_out, idx_ref, w_ref, scale_ref, xa_ref, xb_ref,
                        out_ref, *avs):
    xrefs = (xa_ref, xb_ref)
    for k, av in enumerate(avs):
        for j in range(SUB):
            av[0:N_FEAT, j] = xrefs[k][j].T
        av[N_FEAT] = jnp.zeros((SUB, LANES), jnp.float32)
        av[N_FEAT + 1] = jnp.ones((SUB, LANES), jnp.float32)

    accs = [[jnp.zeros((SUB, LANES), jnp.float32) for _ in range(n_out)]
            for _ in avs]
    for g in range(N_GATES):
        ia = idx_ref[g, 0]
        ib = idx_ref[g, 1]
        rows = [1.0 - av[ia] * av[ib] for av in avs]
        for k, av in enumerate(avs):
            av[N_FEAT + 2 + g] = rows[k]
        for o in range(n_out):
            for k in range(len(avs)):
                accs[k][o] = accs[k][o] + w_ref[g, o] * rows[k]
    for o in range(n_out):
        for k in range(len(avs)):
            s = accs[k][o] * scale_ref[o]
            for j in range(SUB):
                out_ref[o:o + 1, pl.ds(k * BLK + j * LANES, LANES)] = \
                    s[j:j + 1, :]


def kernel(X, gate_weights, output_weights, output_scale):
    n = X.shape[0]
    n_out = output_weights.shape[1]

    info = plsc.get_sparse_core_info()
    nc, ns = 1, info.num_subcores
    gpw = 8

    topk = functools.partial(
        pl.kernel,
        mesh=plsc.VectorSubcoreMesh(
            core_axis_name="c", subcore_axis_name="s", num_cores=1),
        compiler_params=pltpu.CompilerParams(
            needs_layout_passes=False, use_tc_tiling_on_sc=True),
        out_type=jax.ShapeDtypeStruct((N_GATES, 16), jnp.int32),
        scratch_types=[
            pltpu.VMEM((gpw, MAX_CONN), jnp.float32),
            pltpu.VMEM((gpw, 16), jnp.int32),
        ],
    )(functools.partial(_topk_sc_body, nc, gpw))
    idx = topk(gate_weights)

    x3 = X.reshape(n // LANES, LANES, N_FEAT)
    out2 = pl.pallas_call(
        functools.partial(_fused_chain_kernel, n_out),
        grid=(n // (2 * BLK),),
        in_specs=[
            pl.BlockSpec(memory_space=pltpu.SMEM),
            pl.BlockSpec(memory_space=pltpu.SMEM),
            pl.BlockSpec(memory_space=pltpu.SMEM),
            pl.BlockSpec((SUB, LANES, N_FEAT), lambda i: (2 * i, 0, 0)),
            pl.BlockSpec((SUB, LANES, N_FEAT), lambda i: (2 * i + 1, 0, 0)),
        ],
        out_specs=pl.BlockSpec((n_out, 2 * BLK), lambda i: (0, i)),
        out_shape=jax.ShapeDtypeStruct((n_out, n), jnp.float32),
        scratch_shapes=[
            pltpu.VMEM((MAX_CONN, SUB, LANES), jnp.float32)
            for _ in range(NSPLIT)
        ],
    )(idx, output_weights, output_scale, x3, x3)
    return out2.T


# speedup vs baseline: 1.0575x; 1.0575x over previous
import functools

import jax
import jax.numpy as jnp
from jax import lax
from jax.experimental import pallas as pl
from jax.experimental.pallas import tpu as pltpu
from jax.experimental.pallas import tpu_sc as plsc

N_FEAT = 128
N_GATES = 64
MAX_CONN = N_FEAT + 2 + N_GATES
CHUNKS = 13
SUB = 8
LANES = 256
BLK = SUB * LANES


def _topk_sc_body(nc, gpw, gw_hbm, idx_hbm, gw_v, out_v):
    wid = lax.axis_index("s") * nc + lax.axis_index("c")
    big = jnp.int32(1 << 30)
    lane = lax.iota(jnp.int32, 16)

    pltpu.sync_copy(gw_hbm.at[pl.ds(wid * gpw, gpw)], gw_v)

    for j in range(gpw):
        g = wid * gpw + j
        n_valid = N_FEAT + 2 + g

        xs, cols, oks = [], [], []
        for c in range(CHUNKS):
            off = c * 16 if c < 12 else MAX_CONN - 16
            col = lane + off
            ok = (col < n_valid) if c < 12 else ((col >= 192) & (col < n_valid))
            xs.append(gw_v[j, pl.ds(off, 16)])
            cols.append(col)
            oks.append(ok)

        mv = jnp.where(oks[0], xs[0], -1e30)
        for c in range(1, CHUNKS):
            mv = jnp.maximum(mv, jnp.where(oks[c], xs[c], -1e30))
        m = jnp.max(mv)

        es = [jnp.where(oks[c], jnp.exp(xs[c] - m), 0.0) for c in range(CHUNKS)]
        sv = es[0]
        for c in range(1, CHUNKS):
            sv = sv + es[c]
        s = jnp.sum(sv)

        ps = [es[c] / s for c in range(CHUNKS)]

        m1v = jnp.where(oks[0], ps[0], -1.0)
        for c in range(1, CHUNKS):
            m1v = jnp.maximum(m1v, jnp.where(oks[c], ps[c], -1.0))
        m1 = jnp.max(m1v)

        i1v = jnp.where(oks[0] & (ps[0] == m1), cols[0], big)
        for c in range(1, CHUNKS):
            i1v = jnp.minimum(i1v, jnp.where(oks[c] & (ps[c] == m1), cols[c], big))
        i1 = jnp.min(i1v)

        m2v = jnp.where(oks[0] & (cols[0] != i1), ps[0], -1.0)
        for c in range(1, CHUNKS):
            m2v = jnp.maximum(m2v, jnp.where(oks[c] & (cols[c] != i1), ps[c], -1.0))
        m2 = jnp.max(m2v)

        i2v = jnp.where(oks[0] & (cols[0] != i1) & (ps[0] == m2), cols[0], big)
        for c in range(1, CHUNKS):
            i2v = jnp.minimum(
                i2v, jnp.where(oks[c] & (cols[c] != i1) & (ps[c] == m2), cols[c], big))
        i2 = jnp.min(i2v)

        out_v[j, :] = jnp.where(lane == 0, i1, jnp.where(lane == 1, i2, 0))

    pltpu.sync_copy(out_v, idx_hbm.at[pl.ds(wid * gpw, gpw)])


NSPLIT = 2


def _fused_chain_kernel(n_out, idx_ref, w_ref, scale_ref, xa_ref, xb_ref,
                        out_ref, *avs):
    xrefs = (xa_ref, xb_ref)
    for k, av in enumerate(avs):
        for j in range(SUB):
            av[0:N_FEAT, j] = xrefs[k][j].T
        av[N_FEAT] = jnp.zeros((SUB, LANES), jnp.float32)
        av[N_FEAT + 1] = jnp.ones((SUB, LANES), jnp.float32)

    accs = [[jnp.zeros((SUB, LANES), jnp.float32) for _ in range(n_out)]
            for _ in avs]
    for g in range(N_GATES):
        ia = idx_ref[g, 0]
        ib = idx_ref[g, 1]
        rows = [1.0 - av[ia] * av[ib] for av in avs]
        for k, av in enumerate(avs):
            av[N_FEAT + 2 + g] = rows[k]
        for o in range(n_out):
            for k in range(len(avs)):
                accs[k][o] = accs[k][o] + w_ref[g, o] * rows[k]
    for o in range(n_out):
        for k in range(len(avs)):
            s = accs[k][o] * scale_ref[o]
            for j in range(SUB):
                out_ref[o:o + 1, pl.ds(k * BLK + j * LANES, LANES)] = \
                    s[j:j + 1, :]


def kernel(X, gate_weights, output_weights, output_scale):
    n = X.shape[0]
    n_out = output_weights.shape[1]

    info = plsc.get_sparse_core_info()
    nc, ns = 1, info.num_subcores
    gpw = N_GATES // (nc * ns)

    topk = functools.partial(
        pl.kernel,
        mesh=plsc.VectorSubcoreMesh(
            core_axis_name="c", subcore_axis_name="s", num_cores=1),
        compiler_params=pltpu.CompilerParams(needs_layout_passes=False),
        out_type=jax.ShapeDtypeStruct((N_GATES, 16), jnp.int32),
        scratch_types=[
            pltpu.VMEM((gpw, MAX_CONN), jnp.float32),
            pltpu.VMEM((gpw, 16), jnp.int32),
        ],
    )(functools.partial(_topk_sc_body, nc, gpw))
    idx = topk(gate_weights)

    x3 = X.reshape(n // LANES, LANES, N_FEAT)
    out2 = pl.pallas_call(
        functools.partial(_fused_chain_kernel, n_out),
        grid=(n // (2 * BLK),),
        in_specs=[
            pl.BlockSpec(memory_space=pltpu.SMEM),
            pl.BlockSpec(memory_space=pltpu.SMEM),
            pl.BlockSpec(memory_space=pltpu.SMEM),
            pl.BlockSpec((SUB, LANES, N_FEAT), lambda i: (2 * i, 0, 0)),
            pl.BlockSpec((SUB, LANES, N_FEAT), lambda i: (2 * i + 1, 0, 0)),
        ],
        out_specs=pl.BlockSpec((n_out, 2 * BLK), lambda i: (0, i)),
        out_shape=jax.ShapeDtypeStruct((n_out, n), jnp.float32),
        scratch_shapes=[
            pltpu.VMEM((MAX_CONN, SUB, LANES), jnp.float32)
            for _ in range(NSPLIT)
        ],
    )(idx, output_weights, output_scale, x3, x3)
    return out2.T
